# Initial kernel scaffold; baseline (speedup 1.0000x reference)
import jax
import jax.numpy as jnp
from jax.experimental import pallas as pl


def kernel(token_ids, W):
    raise NotImplementedError("write your pallas kernel here")


# speedup vs baseline: 1.2916x; 1.2916x over previous
import functools
import math

import jax
import jax.numpy as jnp
from jax import lax
from jax.experimental import pallas as pl
from jax.experimental.pallas import tpu as pltpu
from jax.experimental.pallas import tpu_sc as plsc

NUM_CORES = 2
NUM_SUBCORES = 16
NUM_WORKERS = NUM_CORES * NUM_SUBCORES
CHUNK = 1024


def kernel(token_ids, W):
    B, S = token_ids.shape
    V, D = W.shape
    N = B * S
    scale = math.sqrt(D)
    n_per_w = N // NUM_WORKERS
    n_chunks = n_per_w // CHUNK
    assert n_chunks * CHUNK * NUM_WORKERS == N

    idx = token_ids.reshape(N).astype(jnp.int32)
    mesh = plsc.VectorSubcoreMesh(core_axis_name="c", subcore_axis_name="s")

    @functools.partial(
        pl.kernel,
        mesh=mesh,
        compiler_params=pltpu.CompilerParams(use_tc_tiling_on_sc=False),
        out_type=jax.ShapeDtypeStruct((N, D), jnp.float32),
        scratch_types=[
            pltpu.VMEM((CHUNK,), jnp.int32),
            pltpu.VMEM((CHUNK, D), jnp.float32),
            pltpu.SemaphoreType.DMA,
        ],
    )
    def emb(idx_hbm, w_hbm, out_hbm, idx_v, rows_v, sem):
        wid = lax.axis_index("s") * NUM_CORES + lax.axis_index("c")
        base = wid * n_per_w

        @pl.loop(0, n_chunks)
        def _(ci):
            cb = base + ci * CHUNK
            pltpu.sync_copy(idx_hbm.at[pl.ds(cb, CHUNK)], idx_v)
            pltpu.async_copy(w_hbm.at[idx_v], rows_v, sem).wait()

            @pl.loop(0, CHUNK)
            def _(r):
                for c in range(0, D, 16):
                    sl = (r, pl.ds(c, 16))
                    rows_v.at[sl][...] = rows_v.at[sl][...] * scale

            pltpu.sync_copy(rows_v, out_hbm.at[pl.ds(cb, CHUNK)])

    out = emb(idx, W)
    return out.reshape(B, S, D)
